# baseline (speedup 1.0000x reference)
import functools
import math

import jax
import jax.numpy as jnp
from jax import lax
from jax.experimental import pallas as pl
from jax.experimental.pallas import tpu as pltpu
from jax.experimental.pallas import tpu_sc as plsc

_EMB = 64
_SCALE = math.sqrt(_EMB)
_SPC = 2
_GROUPS = ((0, 128), (128, 72))


@functools.lru_cache(maxsize=None)
def _build(batch, seq):
    info = plsc.get_sparse_core_info()
    nw = info.num_cores * info.num_subcores
    assert batch % (nw * 2 * _SPC) == 0 and seq == sum(n for _, n in _GROUPS)
    bpw = batch // nw
    chunks = bpw // _SPC
    mesh = plsc.VectorSubcoreMesh(core_axis_name="c", subcore_axis_name="s")

    @functools.partial(
        pl.kernel,
        mesh=mesh,
        compiler_params=pltpu.CompilerParams(use_tc_tiling_on_sc=False),
        out_type=jax.ShapeDtypeStruct((batch, seq, _EMB), jnp.float32),
        scratch_types=[
            pltpu.VMEM((2, _SPC, seq), jnp.int32),
            pltpu.VMEM((2, _SPC, seq, _EMB), jnp.float32),
            [pltpu.SemaphoreType.DMA] * 2,
            [pltpu.SemaphoreType.DMA] * 2,
            [pltpu.SemaphoreType.DMA] * 2,
        ],
    )
    def embed(idx_hbm, table_hbm, out_hbm, idx_v, rows_v, gsem, ssem, isem):
        wid = lax.axis_index("s") * info.num_cores + lax.axis_index("c")
        base = wid * bpw

        def fire_idx(chunk, p):
            pltpu.make_async_copy(
                idx_hbm.at[pl.ds(base + chunk * _SPC, _SPC)], idx_v.at[p], isem[p]
            ).start()

        def fire_gathers(p):
            for r in range(_SPC):
                for off, n in _GROUPS:
                    pltpu.make_async_copy(
                        table_hbm.at[idx_v.at[p, r, pl.ds(off, n)]],
                        rows_v.at[p, r, pl.ds(off, n)],
                        gsem[p],
                    ).start()

        def drain_gathers(p):
            for r in range(_SPC):
                for off, n in _GROUPS:
                    pltpu.make_async_copy(
                        table_hbm.at[idx_v.at[p, r, pl.ds(off, n)]],
                        rows_v.at[p, r, pl.ds(off, n)],
                        gsem[p],
                    ).wait()

        def scatter(chunk, p, wait):
            cp = pltpu.make_async_copy(
                rows_v.at[p], out_hbm.at[pl.ds(base + chunk * _SPC, _SPC)],
                ssem[p],
            )
            cp.wait() if wait else cp.start()

        def step(chunk, p):
            q = 1 - p
            drain_gathers(p)
            @pl.when(jnp.logical_and(chunk > 0, chunk + 1 < chunks))
            def _():
                scatter(chunk - 1, q, wait=True)

            @pl.when(chunk + 1 < chunks)
            def _():
                pltpu.make_async_copy(
                    idx_hbm.at[pl.ds(base + (chunk + 1) * _SPC, _SPC)],
                    idx_v.at[q], isem[q],
                ).wait()
                fire_gathers(q)

            @pl.when(chunk + 2 < chunks)
            def _():
                fire_idx(chunk + 2, p)

            @plsc.parallel_loop(0, seq, unroll=4)
            def _(i):
                for r in range(_SPC):
                    for k in range(_EMB // 16):
                        sl = pl.ds(k * 16, 16)
                        rows_v[p, r, i, sl] = rows_v[p, r, i, sl] * _SCALE

            scatter(chunk, p, wait=False)

        pltpu.sync_copy(idx_hbm.at[pl.ds(base, _SPC)], idx_v.at[0])
        fire_gathers(0)
        fire_idx(1, 1)

        def pair(h, carry):
            step(2 * h, 0)
            step(2 * h + 1, 1)
            return carry

        lax.fori_loop(0, chunks // 2, pair, 0)
        scatter(chunks - 2, 0, wait=True)
        scatter(chunks - 1, 1, wait=True)

    return embed


@jax.jit
def kernel(tokens, embedding_weight):
    batch, seq = tokens.shape
    return _build(batch, seq)(tokens.astype(jnp.int32), embedding_weight)


# speedup vs baseline: 1.0979x; 1.0979x over previous
import functools
import math

import jax
import jax.numpy as jnp
from jax import lax
from jax.experimental import pallas as pl
from jax.experimental.pallas import tpu as pltpu
from jax.experimental.pallas import tpu_sc as plsc

_EMB = 64
_SCALE = math.sqrt(_EMB)
_GROUPS = ((0, 128), (128, 72))


@functools.lru_cache(maxsize=None)
def _build(batch, seq):
    info = plsc.get_sparse_core_info()
    nl = info.num_lanes
    nw = info.num_cores * info.num_subcores
    assert batch % (nw * 2) == 0 and seq == sum(n for _, n in _GROUPS)
    bpw = batch // nw
    mesh = plsc.VectorSubcoreMesh(core_axis_name="c", subcore_axis_name="s")

    @functools.partial(
        pl.kernel,
        mesh=mesh,
        out_type=jax.ShapeDtypeStruct((batch, seq, _EMB), jnp.float32),
        scratch_types=[
            pltpu.VMEM((2, 1, seq), jnp.int32),
            pltpu.VMEM((2, seq, 2 * _EMB), jnp.float32),
            pltpu.VMEM((2, 1, seq, _EMB), jnp.float32),
            [pltpu.SemaphoreType.DMA] * 2,
            [pltpu.SemaphoreType.DMA] * 2,
            [pltpu.SemaphoreType.DMA] * 2,
        ],
    )
    def embed(idx_hbm, wide_hbm, out_hbm, idx_v, rows_v, outs_v,
              gsem, ssem, isem):
        wid = lax.axis_index("s") * info.num_cores + lax.axis_index("c")
        base = wid * bpw

        def fire_idx(chunk, p):
            pltpu.make_async_copy(
                idx_hbm.at[pl.ds(base + chunk, 1)], idx_v.at[p], isem[p]
            ).start()

        def fire_gathers(p):
            for off, n in _GROUPS:
                pltpu.make_async_copy(
                    wide_hbm.at[idx_v.at[p, 0, pl.ds(off, n)]],
                    rows_v.at[p, pl.ds(off, n)],
                    gsem[p],
                ).start()

        def drain_gathers(p):
            for off, n in _GROUPS:
                pltpu.make_async_copy(
                    wide_hbm.at[idx_v.at[p, 0, pl.ds(off, n)]],
                    rows_v.at[p, pl.ds(off, n)],
                    gsem[p],
                ).wait()

        def scatter(chunk, p, wait):
            cp = pltpu.make_async_copy(
                outs_v.at[p], out_hbm.at[pl.ds(base + chunk, 1)], ssem[p],
            )
            cp.wait() if wait else cp.start()

        def step(chunk, p):
            q = 1 - p
            drain_gathers(p)
            @pl.when(jnp.logical_and(chunk > 0, chunk + 1 < bpw))
            def _():
                scatter(chunk - 1, q, wait=True)

            @pl.when(chunk + 1 < bpw)
            def _():
                pltpu.make_async_copy(
                    idx_hbm.at[pl.ds(base + chunk + 1, 1)], idx_v.at[q], isem[q],
                ).wait()
                fire_gathers(q)

            @pl.when(chunk + 2 < bpw)
            def _():
                fire_idx(chunk + 2, p)

            @plsc.parallel_loop(0, seq, unroll=4)
            def _(i):
                for k in range(_EMB // nl):
                    sl = pl.ds(k * nl, nl)
                    outs_v[p, 0, i, sl] = rows_v[p, i, sl] * _SCALE

            scatter(chunk, p, wait=False)

        pltpu.sync_copy(idx_hbm.at[pl.ds(base, 1)], idx_v.at[0])
        fire_gathers(0)
        fire_idx(1, 1)

        def pair(h, carry):
            step(2 * h, 0)
            step(2 * h + 1, 1)
            return carry

        lax.fori_loop(0, bpw // 2, pair, 0)
        scatter(bpw - 2, 0, wait=True)
        scatter(bpw - 1, 1, wait=True)

    return embed


@jax.jit
def kernel(tokens, embedding_weight):
    batch, seq = tokens.shape
    vocab, emb = embedding_weight.shape
    wide = jnp.pad(embedding_weight, ((0, 0), (0, emb)))
    return _build(batch, seq)(tokens.astype(jnp.int32), wide)
